# Initial kernel scaffold; baseline (speedup 1.0000x reference)
import jax
import jax.numpy as jnp
from jax.experimental import pallas as pl


def kernel(emb, all_emb, r_mask, node_src, edge_dst, relation_ids):
    raise NotImplementedError("write your pallas kernel here")


# speedup vs baseline: 4.5173x; 4.5173x over previous
import functools

import jax
import jax.numpy as jnp
from jax import lax
from jax.experimental import pallas as pl
from jax.experimental.pallas import tpu as pltpu
from jax.experimental.pallas import tpu_sc as plsc

N = 10000
M = 160000
R = 64
D = 128
BS = 128

NB = 2048
EB = 16000
CHUNK = 2000
NCHUNK = M // CHUNK
COLS_PER_TILE = BS // 32


def _matmul_body(emb_ref, ae_ref, out_ref):
    out_ref[...] = jnp.tanh(lax.dot_general(
        emb_ref[...], ae_ref[...],
        dimension_numbers=(((1,), (1,)), ((), ())),
        preferred_element_type=jnp.float32))


def _pack_body(rmask_ref, h_ref, out_ref):
    rv = lax.broadcasted_iota(jnp.int32, (R, 1), 0).astype(jnp.float32) * 65536.0
    s = jnp.sum(rmask_ref[...] * rv, axis=0, keepdims=True)
    out_ref[...] = s.astype(jnp.int32) + h_ref[...]


def _sc_edge_body(disT, w_hbm, t_hbm, relids_hbm, out_hbm,
                  relids_v, dis_cols, acc, w_buf, t_buf):
    wid = lax.axis_index("s") * 2 + lax.axis_index("c")
    col0 = wid * COLS_PER_TILE

    pltpu.sync_copy(relids_hbm, relids_v)
    for j in range(COLS_PER_TILE):
        pltpu.sync_copy(disT.at[col0 + j], dis_cols.at[j])

    zero16 = jnp.zeros((16,), jnp.float32)

    def zero_body(i, carry):
        for j in range(COLS_PER_TILE):
            acc[j, pl.ds(i * 16, 16)] = zero16
        return carry

    lax.fori_loop(0, N // 16, zero_body, 0)

    rel_vecs = []
    cidx = []
    for j in range(COLS_PER_TILE):
        c = col0 + j
        rel_vecs.append(plsc.load_gather(
            relids_v, [jnp.full((16,), c // 16, jnp.int32),
                       jnp.full((16,), c % 16, jnp.int32)]))
        cidx.append(jnp.full((16,), j, jnp.int32))

    def chunk_body(g, carry):
        cg = lax.rem(g + wid * 2, NCHUNK)
        base = cg * CHUNK
        pltpu.sync_copy(w_hbm.at[pl.ds(base, CHUNK)], w_buf)
        pltpu.sync_copy(t_hbm.at[pl.ds(base, CHUNK)], t_buf)

        def iter_body(i, c2):
            w_vec = w_buf[pl.ds(i * 16, 16)]
            t_vec = t_buf[pl.ds(i * 16, 16)]
            r_vec = lax.shift_right_logical(w_vec, 16)
            h_vec = lax.bitwise_and(w_vec, 0xFFFF)
            for j in range(COLS_PER_TILE):
                m = r_vec == rel_vecs[j]
                val = plsc.load_gather(dis_cols, [cidx[j], h_vec])
                plsc.addupdate_scatter(acc, [cidx[j], t_vec], val, mask=m)
            return c2

        lax.fori_loop(0, CHUNK // 16, iter_body, 0)
        return carry

    lax.fori_loop(0, NCHUNK, chunk_body, 0)

    for j in range(COLS_PER_TILE):
        pltpu.sync_copy(acc.at[j], out_hbm.at[col0 + j])


_sc_edge_kernel = functools.partial(
    pl.kernel,
    out_type=jax.ShapeDtypeStruct((BS, N), jnp.float32),
    mesh=plsc.VectorSubcoreMesh(core_axis_name="c", subcore_axis_name="s"),
    compiler_params=pltpu.CompilerParams(needs_layout_passes=False),
    scratch_types=[
        pltpu.VMEM((BS // 16, 16), jnp.int32),
        pltpu.VMEM((COLS_PER_TILE, N), jnp.float32),
        pltpu.VMEM((COLS_PER_TILE, N), jnp.float32),
        pltpu.VMEM((CHUNK,), jnp.int32),
        pltpu.VMEM((CHUNK,), jnp.int32),
    ],
)(_sc_edge_body)


def kernel(emb, all_emb, r_mask, node_src, edge_dst, relation_ids):
    node_src = node_src.astype(jnp.int32)
    edge_dst = edge_dst.astype(jnp.int32)
    relation_ids = relation_ids.astype(jnp.int32)

    disT = pl.pallas_call(
        _matmul_body,
        grid=(pl.cdiv(N, NB),),
        in_specs=[
            pl.BlockSpec((BS, D), lambda i: (0, 0)),
            pl.BlockSpec((NB, D), lambda i: (i, 0)),
        ],
        out_specs=pl.BlockSpec((BS, NB), lambda i: (0, i)),
        out_shape=jax.ShapeDtypeStruct((BS, N), jnp.float32),
    )(emb, all_emb)

    w = pl.pallas_call(
        _pack_body,
        grid=(M // EB,),
        in_specs=[
            pl.BlockSpec((R, EB), lambda i: (0, i)),
            pl.BlockSpec((1, EB), lambda i: (0, i)),
        ],
        out_specs=pl.BlockSpec((1, EB), lambda i: (0, i)),
        out_shape=jax.ShapeDtypeStruct((1, M), jnp.int32),
    )(r_mask, node_src.reshape(1, M)).reshape(M)

    return _sc_edge_kernel(disT, w, edge_dst,
                           relation_ids.reshape(BS // 16, 16))


# speedup vs baseline: 6.2273x; 1.3786x over previous
import functools

import jax
import jax.numpy as jnp
from jax import lax
from jax.experimental import pallas as pl
from jax.experimental.pallas import tpu as pltpu
from jax.experimental.pallas import tpu_sc as plsc

N = 10000
M = 160000
R = 64
D = 128
BS = 128

NB = 2048
EB = 16000
CHUNK = 4000
NCHUNK = M // CHUNK
NPAIR = NCHUNK // 2
COLS_PER_TILE = BS // 32


def _matmul_body(emb_ref, ae_ref, out_ref):
    out_ref[...] = jnp.tanh(lax.dot_general(
        emb_ref[...], ae_ref[...],
        dimension_numbers=(((1,), (1,)), ((), ())),
        preferred_element_type=jnp.float32))


def _pack_body(rmask_ref, h_ref, out_ref):
    rv = lax.broadcasted_iota(jnp.int32, (R, 1), 0).astype(jnp.float32) * 65536.0
    s = jnp.sum(rmask_ref[...] * rv, axis=0, keepdims=True)
    out_ref[...] = s.astype(jnp.int32) + h_ref[...]


def _sc_edge_body(disT, w_hbm, t_hbm, relids_hbm, out_hbm,
                  relids_v, dis_cols, acc, w_bufa, t_bufa, w_bufb, t_bufb,
                  sema, semb):
    wid = lax.axis_index("s") * 2 + lax.axis_index("c")
    col0 = wid * COLS_PER_TILE

    pltpu.sync_copy(relids_hbm, relids_v)
    for j in range(COLS_PER_TILE):
        pltpu.sync_copy(disT.at[col0 + j], dis_cols.at[j])

    zero16 = jnp.zeros((16,), jnp.float32)

    def zero_body(i, carry):
        for j in range(COLS_PER_TILE):
            acc[j, pl.ds(i * 16, 16)] = zero16
        return carry

    lax.fori_loop(0, N // 16, zero_body, 0)

    rel_vecs = []
    cidx = []
    for j in range(COLS_PER_TILE):
        c = col0 + j
        rel_vecs.append(plsc.load_gather(
            relids_v, [jnp.full((16,), c // 16, jnp.int32),
                       jnp.full((16,), c % 16, jnp.int32)]))
        cidx.append(jnp.full((16,), j, jnp.int32))

    def start_chunk(k, w_dst, t_dst, sem):
        cg = lax.rem(k + 2 * wid, NCHUNK)
        base = cg * CHUNK
        pltpu.async_copy(w_hbm.at[pl.ds(base, CHUNK)], w_dst, sem)
        pltpu.async_copy(t_hbm.at[pl.ds(base, CHUNK)], t_dst, sem)

    def wait_chunk(w_dst, t_dst, sem):
        pltpu.make_async_copy(w_hbm.at[pl.ds(0, CHUNK)], w_dst, sem).wait()
        pltpu.make_async_copy(t_hbm.at[pl.ds(0, CHUNK)], t_dst, sem).wait()

    def process(w_src, t_src):
        def iter_body(i, c2):
            w_vec = w_src[pl.ds(i * 16, 16)]
            t_vec = t_src[pl.ds(i * 16, 16)]
            r_vec = lax.shift_right_logical(w_vec, 16)
            h_vec = lax.bitwise_and(w_vec, 0xFFFF)
            for j in range(COLS_PER_TILE):
                m = r_vec == rel_vecs[j]
                val = plsc.load_gather(dis_cols, [cidx[j], h_vec])
                plsc.addupdate_scatter(acc, [cidx[j], t_vec], val, mask=m)
            return c2

        lax.fori_loop(0, CHUNK // 16, iter_body, 0, unroll=8)

    start_chunk(0, w_bufa, t_bufa, sema)
    start_chunk(1, w_bufb, t_bufb, semb)

    def pair_body(u, carry):
        wait_chunk(w_bufa, t_bufa, sema)
        process(w_bufa, t_bufa)

        @pl.when(u < NPAIR - 1)
        def _():
            start_chunk(2 * u + 2, w_bufa, t_bufa, sema)

        wait_chunk(w_bufb, t_bufb, semb)
        process(w_bufb, t_bufb)

        @pl.when(u < NPAIR - 1)
        def _():
            start_chunk(2 * u + 3, w_bufb, t_bufb, semb)

        return carry

    lax.fori_loop(0, NPAIR, pair_body, 0)

    for j in range(COLS_PER_TILE):
        pltpu.sync_copy(acc.at[j], out_hbm.at[col0 + j])


_sc_edge_kernel = functools.partial(
    pl.kernel,
    out_type=jax.ShapeDtypeStruct((BS, N), jnp.float32),
    mesh=plsc.VectorSubcoreMesh(core_axis_name="c", subcore_axis_name="s"),
    compiler_params=pltpu.CompilerParams(needs_layout_passes=False),
    scratch_types=[
        pltpu.VMEM((BS // 16, 16), jnp.int32),
        pltpu.VMEM((COLS_PER_TILE, N), jnp.float32),
        pltpu.VMEM((COLS_PER_TILE, N), jnp.float32),
        pltpu.VMEM((CHUNK,), jnp.int32),
        pltpu.VMEM((CHUNK,), jnp.int32),
        pltpu.VMEM((CHUNK,), jnp.int32),
        pltpu.VMEM((CHUNK,), jnp.int32),
        pltpu.SemaphoreType.DMA,
        pltpu.SemaphoreType.DMA,
    ],
)(_sc_edge_body)


def kernel(emb, all_emb, r_mask, node_src, edge_dst, relation_ids):
    node_src = node_src.astype(jnp.int32)
    edge_dst = edge_dst.astype(jnp.int32)
    relation_ids = relation_ids.astype(jnp.int32)

    disT = pl.pallas_call(
        _matmul_body,
        grid=(pl.cdiv(N, NB),),
        in_specs=[
            pl.BlockSpec((BS, D), lambda i: (0, 0)),
            pl.BlockSpec((NB, D), lambda i: (i, 0)),
        ],
        out_specs=pl.BlockSpec((BS, NB), lambda i: (0, i)),
        out_shape=jax.ShapeDtypeStruct((BS, N), jnp.float32),
    )(emb, all_emb)

    w = pl.pallas_call(
        _pack_body,
        grid=(M // EB,),
        in_specs=[
            pl.BlockSpec((R, EB), lambda i: (0, i)),
            pl.BlockSpec((1, EB), lambda i: (0, i)),
        ],
        out_specs=pl.BlockSpec((1, EB), lambda i: (0, i)),
        out_shape=jax.ShapeDtypeStruct((1, M), jnp.int32),
    )(r_mask, node_src.reshape(1, M)).reshape(M)

    return _sc_edge_kernel(disT, w, edge_dst,
                           relation_ids.reshape(BS // 16, 16))


# speedup vs baseline: 14.9821x; 2.4059x over previous
import functools

import jax
import jax.numpy as jnp
from jax import lax
from jax.experimental import pallas as pl
from jax.experimental.pallas import tpu as pltpu
from jax.experimental.pallas import tpu_sc as plsc

N = 10000
M = 160000
R = 64
D = 128
BS = 128

NB = 2048
EB = 16000
CHUNK = 4000
NCHUNK = M // CHUNK
NPAIR = NCHUNK // 2
NTILE = 32
COLS_PER_TILE = BS // NTILE


def _matmul_body(emb_ref, ae_ref, out_ref):
    out_ref[...] = jnp.tanh(lax.dot_general(
        emb_ref[...], ae_ref[...],
        dimension_numbers=(((1,), (1,)), ((), ())),
        preferred_element_type=jnp.float32))


def _pack_body(rmask_ref, h_ref, out_ref):
    rv = lax.broadcasted_iota(jnp.int32, (R, 1), 0).astype(jnp.float32) * 65536.0
    s = jnp.sum(rmask_ref[...] * rv, axis=0, keepdims=True)
    out_ref[...] = s.astype(jnp.int32) + h_ref[...]


def _sc_edge_body(disT, w_hbm, t_hbm, perm_hbm, qrel_hbm, out_hbm,
                  perm_v, qrel_v, jtab, dis_cols, acc,
                  w_bufa, t_bufa, w_bufb, t_bufb, sema, semb):
    wid = lax.axis_index("s") * 2 + lax.axis_index("c")
    row0 = wid * COLS_PER_TILE

    pltpu.sync_copy(perm_hbm, perm_v)
    pltpu.sync_copy(qrel_hbm, qrel_v)
    for j in range(COLS_PER_TILE):
        pltpu.sync_copy(disT.at[row0 + j], dis_cols.at[j])

    zero16 = jnp.zeros((16,), jnp.float32)

    def zero_body(i, carry):
        for j in range(COLS_PER_TILE):
            acc[j, pl.ds(i * 16, 16)] = zero16
        return carry

    lax.fori_loop(0, N // 16, zero_body, 0)

    widv = jnp.full((16,), wid, jnp.int32)
    rel_vecs = [plsc.load_gather(qrel_v, [widv, jnp.full((16,), j, jnp.int32)])
                for j in range(COLS_PER_TILE)]
    cidx = [jnp.full((16,), j, jnp.int32) for j in range(COLS_PER_TILE)]
    z16 = jnp.zeros((16,), jnp.int32)
    miss16 = jnp.full((16,), COLS_PER_TILE, jnp.int32)
    lane0 = lax.broadcasted_iota(jnp.int32, (16,), 0) == z16

    for i in range(R // 16):
        jtab[0, pl.ds(i * 16, 16)] = miss16
    for j in range(COLS_PER_TILE):
        plsc.store_scatter(jtab, [z16, rel_vecs[j]], cidx[j], mask=lane0)

    dup = jnp.zeros((16,), jnp.int32)
    for a in range(COLS_PER_TILE):
        for b in range(a + 1, COLS_PER_TILE):
            dup = dup + (rel_vecs[a] == rel_vecs[b]).astype(jnp.int32)
    has_dup = jnp.sum(dup, axis=0) > 0

    def start_chunk(k, w_dst, t_dst, sem):
        cg = lax.rem(k + 2 * wid, NCHUNK)
        base = cg * CHUNK
        pltpu.async_copy(w_hbm.at[pl.ds(base, CHUNK)], w_dst, sem)
        pltpu.async_copy(t_hbm.at[pl.ds(base, CHUNK)], t_dst, sem)

    def wait_chunk(w_dst, t_dst, sem):
        pltpu.make_async_copy(w_hbm.at[pl.ds(0, CHUNK)], w_dst, sem).wait()
        pltpu.make_async_copy(t_hbm.at[pl.ds(0, CHUNK)], t_dst, sem).wait()

    def process_fast(w_src, t_src):
        @plsc.parallel_loop(0, CHUNK // 16, unroll=8)
        def iter_body(i):
            w_vec = w_src[pl.ds(i * 16, 16)]
            t_vec = t_src[pl.ds(i * 16, 16)]
            r_vec = lax.shift_right_logical(w_vec, 16)
            h_vec = lax.bitwise_and(w_vec, 0xFFFF)
            jsel = plsc.load_gather(jtab, [z16, r_vec])
            m = jsel != miss16
            val = plsc.load_gather(dis_cols, [jsel, h_vec], mask=m)
            plsc.addupdate_scatter(acc, [jsel, t_vec], val, mask=m)

    def process_slow(w_src, t_src):
        @plsc.parallel_loop(0, CHUNK // 16, unroll=4)
        def iter_body(i):
            w_vec = w_src[pl.ds(i * 16, 16)]
            t_vec = t_src[pl.ds(i * 16, 16)]
            r_vec = lax.shift_right_logical(w_vec, 16)
            h_vec = lax.bitwise_and(w_vec, 0xFFFF)
            for j in range(COLS_PER_TILE):
                m = r_vec == rel_vecs[j]
                val = plsc.load_gather(dis_cols, [cidx[j], h_vec])
                plsc.addupdate_scatter(acc, [cidx[j], t_vec], val, mask=m)

    start_chunk(0, w_bufa, t_bufa, sema)
    start_chunk(1, w_bufb, t_bufb, semb)

    def make_pair_body(process):
        def pair_body(u, carry):
            wait_chunk(w_bufa, t_bufa, sema)
            process(w_bufa, t_bufa)

            @pl.when(u < NPAIR - 1)
            def _():
                start_chunk(2 * u + 2, w_bufa, t_bufa, sema)

            wait_chunk(w_bufb, t_bufb, semb)
            process(w_bufb, t_bufb)

            @pl.when(u < NPAIR - 1)
            def _():
                start_chunk(2 * u + 3, w_bufb, t_bufb, semb)

            return carry
        return pair_body

    @pl.when(jnp.logical_not(has_dup))
    def _():
        lax.fori_loop(0, NPAIR, make_pair_body(process_fast), 0)

    @pl.when(has_dup)
    def _():
        lax.fori_loop(0, NPAIR, make_pair_body(process_slow), 0)

    pltpu.async_copy(acc, out_hbm.at[perm_v.at[wid]], sema).wait()


_sc_edge_kernel = functools.partial(
    pl.kernel,
    out_type=jax.ShapeDtypeStruct((BS, N), jnp.float32),
    mesh=plsc.VectorSubcoreMesh(core_axis_name="c", subcore_axis_name="s"),
    compiler_params=pltpu.CompilerParams(needs_layout_passes=False,
                                         use_tc_tiling_on_sc=False),
    scratch_types=[
        pltpu.VMEM((NTILE, COLS_PER_TILE), jnp.int32),
        pltpu.VMEM((NTILE, COLS_PER_TILE), jnp.int32),
        pltpu.VMEM((1, R), jnp.int32),
        pltpu.VMEM((COLS_PER_TILE, N), jnp.float32),
        pltpu.VMEM((COLS_PER_TILE, N), jnp.float32),
        pltpu.VMEM((CHUNK,), jnp.int32),
        pltpu.VMEM((CHUNK,), jnp.int32),
        pltpu.VMEM((CHUNK,), jnp.int32),
        pltpu.VMEM((CHUNK,), jnp.int32),
        pltpu.SemaphoreType.DMA,
        pltpu.SemaphoreType.DMA,
    ],
)(_sc_edge_body)


def kernel(emb, all_emb, r_mask, node_src, edge_dst, relation_ids):
    node_src = node_src.astype(jnp.int32)
    edge_dst = edge_dst.astype(jnp.int32)
    relation_ids = relation_ids.astype(jnp.int32)

    order = jnp.argsort(relation_ids).astype(jnp.int32)
    rowperm = order.reshape(COLS_PER_TILE, NTILE).T.reshape(BS)
    perm = rowperm.reshape(NTILE, COLS_PER_TILE)
    qrel = jnp.take(relation_ids, perm, axis=0)
    emb = jnp.take(emb, rowperm, axis=0)

    disT = pl.pallas_call(
        _matmul_body,
        grid=(pl.cdiv(N, NB),),
        in_specs=[
            pl.BlockSpec((BS, D), lambda i: (0, 0)),
            pl.BlockSpec((NB, D), lambda i: (i, 0)),
        ],
        out_specs=pl.BlockSpec((BS, NB), lambda i: (0, i)),
        out_shape=jax.ShapeDtypeStruct((BS, N), jnp.float32),
    )(emb, all_emb)

    w = pl.pallas_call(
        _pack_body,
        grid=(M // EB,),
        in_specs=[
            pl.BlockSpec((R, EB), lambda i: (0, i)),
            pl.BlockSpec((1, EB), lambda i: (0, i)),
        ],
        out_specs=pl.BlockSpec((1, EB), lambda i: (0, i)),
        out_shape=jax.ShapeDtypeStruct((1, M), jnp.int32),
    )(r_mask, node_src.reshape(1, M)).reshape(M)

    return _sc_edge_kernel(disT, w, edge_dst, perm, qrel)


# speedup vs baseline: 17.3134x; 1.1556x over previous
import functools

import jax
import jax.numpy as jnp
from jax import lax
from jax.experimental import pallas as pl
from jax.experimental.pallas import tpu as pltpu
from jax.experimental.pallas import tpu_sc as plsc

N = 10000
M = 160000
R = 64
D = 128
BS = 128

NB = 1024
EB = 16384
CHUNK = 8000
NCHUNK = M // CHUNK
NPAIR = NCHUNK // 2
NTILE = 32
COLS_PER_TILE = BS // NTILE


def _tc_body(emb_ref, ae_ref, rmask_ref, h_ref, dis_ref, w_ref):
    dis_ref[...] = jnp.tanh(lax.dot_general(
        emb_ref[...], ae_ref[...],
        dimension_numbers=(((1,), (1,)), ((), ())),
        preferred_element_type=jnp.float32))
    rv = lax.broadcasted_iota(jnp.int32, (R, 1), 0).astype(jnp.float32) * 65536.0
    s = jnp.sum(rmask_ref[...] * rv, axis=0)
    w_ref[...] = s.astype(jnp.int32) + h_ref[...]


def _sc_edge_body(disT, w_hbm, t_hbm, perm_hbm, qrel_hbm, out_hbm,
                  perm_v, qrel_v, jtab, dis_cols, acc,
                  w_bufa, t_bufa, w_bufb, t_bufb, sema, semb):
    wid = lax.axis_index("s") * 2 + lax.axis_index("c")
    row0 = wid * COLS_PER_TILE

    pltpu.sync_copy(perm_hbm, perm_v)
    pltpu.sync_copy(qrel_hbm, qrel_v)
    for j in range(COLS_PER_TILE):
        pltpu.sync_copy(disT.at[row0 + j], dis_cols.at[j])

    zero16 = jnp.zeros((16,), jnp.float32)

    def zero_body(i, carry):
        for j in range(COLS_PER_TILE):
            acc[j, pl.ds(i * 16, 16)] = zero16
        return carry

    lax.fori_loop(0, N // 16, zero_body, 0)

    widv = jnp.full((16,), wid, jnp.int32)
    rel_vecs = [plsc.load_gather(qrel_v, [widv, jnp.full((16,), j, jnp.int32)])
                for j in range(COLS_PER_TILE)]
    cidx = [jnp.full((16,), j, jnp.int32) for j in range(COLS_PER_TILE)]
    z16 = jnp.zeros((16,), jnp.int32)
    miss16 = jnp.full((16,), COLS_PER_TILE, jnp.int32)
    lane0 = lax.broadcasted_iota(jnp.int32, (16,), 0) == z16

    for i in range(R // 16):
        jtab[0, pl.ds(i * 16, 16)] = miss16
    for j in range(COLS_PER_TILE):
        plsc.store_scatter(jtab, [z16, rel_vecs[j]], cidx[j], mask=lane0)

    dup = jnp.zeros((16,), jnp.int32)
    for a in range(COLS_PER_TILE):
        for b in range(a + 1, COLS_PER_TILE):
            dup = dup + (rel_vecs[a] == rel_vecs[b]).astype(jnp.int32)
    has_dup = jnp.sum(dup, axis=0) > 0

    def start_chunk(k, w_dst, t_dst, sem):
        cg = lax.rem(k + 2 * wid, NCHUNK)
        base = cg * CHUNK
        pltpu.async_copy(w_hbm.at[pl.ds(base, CHUNK)], w_dst, sem)
        pltpu.async_copy(t_hbm.at[pl.ds(base, CHUNK)], t_dst, sem)

    def wait_chunk(w_dst, t_dst, sem):
        pltpu.make_async_copy(w_hbm.at[pl.ds(0, CHUNK)], w_dst, sem).wait()
        pltpu.make_async_copy(t_hbm.at[pl.ds(0, CHUNK)], t_dst, sem).wait()

    def process_fast(w_src, t_src):
        @plsc.parallel_loop(0, CHUNK // 16, unroll=8)
        def iter_body(i):
            w_vec = w_src[pl.ds(i * 16, 16)]
            t_vec = t_src[pl.ds(i * 16, 16)]
            r_vec = lax.shift_right_logical(w_vec, 16)
            h_vec = lax.bitwise_and(w_vec, 0xFFFF)
            jsel = plsc.load_gather(jtab, [z16, r_vec])
            m = jsel != miss16
            val = plsc.load_gather(dis_cols, [jsel, h_vec], mask=m)
            plsc.addupdate_scatter(acc, [jsel, t_vec], val, mask=m)

    def process_slow(w_src, t_src):
        @plsc.parallel_loop(0, CHUNK // 16, unroll=4)
        def iter_body(i):
            w_vec = w_src[pl.ds(i * 16, 16)]
            t_vec = t_src[pl.ds(i * 16, 16)]
            r_vec = lax.shift_right_logical(w_vec, 16)
            h_vec = lax.bitwise_and(w_vec, 0xFFFF)
            for j in range(COLS_PER_TILE):
                m = r_vec == rel_vecs[j]
                val = plsc.load_gather(dis_cols, [cidx[j], h_vec])
                plsc.addupdate_scatter(acc, [cidx[j], t_vec], val, mask=m)

    start_chunk(0, w_bufa, t_bufa, sema)
    start_chunk(1, w_bufb, t_bufb, semb)

    def make_pair_body(process):
        def pair_body(u, carry):
            wait_chunk(w_bufa, t_bufa, sema)
            process(w_bufa, t_bufa)

            @pl.when(u < NPAIR - 1)
            def _():
                start_chunk(2 * u + 2, w_bufa, t_bufa, sema)

            wait_chunk(w_bufb, t_bufb, semb)
            process(w_bufb, t_bufb)

            @pl.when(u < NPAIR - 1)
            def _():
                start_chunk(2 * u + 3, w_bufb, t_bufb, semb)

            return carry
        return pair_body

    @pl.when(jnp.logical_not(has_dup))
    def _():
        lax.fori_loop(0, NPAIR, make_pair_body(process_fast), 0)

    @pl.when(has_dup)
    def _():
        lax.fori_loop(0, NPAIR, make_pair_body(process_slow), 0)

    pltpu.async_copy(acc, out_hbm.at[perm_v.at[wid]], sema).wait()


_sc_edge_kernel = functools.partial(
    pl.kernel,
    out_type=jax.ShapeDtypeStruct((BS, N), jnp.float32),
    mesh=plsc.VectorSubcoreMesh(core_axis_name="c", subcore_axis_name="s"),
    compiler_params=pltpu.CompilerParams(needs_layout_passes=False,
                                         use_tc_tiling_on_sc=False),
    scratch_types=[
        pltpu.VMEM((NTILE, COLS_PER_TILE), jnp.int32),
        pltpu.VMEM((NTILE, COLS_PER_TILE), jnp.int32),
        pltpu.VMEM((1, R), jnp.int32),
        pltpu.VMEM((COLS_PER_TILE, N), jnp.float32),
        pltpu.VMEM((COLS_PER_TILE, N), jnp.float32),
        pltpu.VMEM((CHUNK,), jnp.int32),
        pltpu.VMEM((CHUNK,), jnp.int32),
        pltpu.VMEM((CHUNK,), jnp.int32),
        pltpu.VMEM((CHUNK,), jnp.int32),
        pltpu.SemaphoreType.DMA,
        pltpu.SemaphoreType.DMA,
    ],
)(_sc_edge_body)


def kernel(emb, all_emb, r_mask, node_src, edge_dst, relation_ids):
    node_src = node_src.astype(jnp.int32)
    edge_dst = edge_dst.astype(jnp.int32)
    relation_ids = relation_ids.astype(jnp.int32)

    order = jnp.argsort(relation_ids).astype(jnp.int32)
    rowperm = order.reshape(COLS_PER_TILE, NTILE).T.reshape(BS)
    perm = rowperm.reshape(NTILE, COLS_PER_TILE)
    qrel = jnp.take(relation_ids, perm, axis=0)
    emb = jnp.take(emb, rowperm, axis=0)

    disT, w = pl.pallas_call(
        _tc_body,
        grid=(pl.cdiv(M, EB),),
        in_specs=[
            pl.BlockSpec((BS, D), lambda i: (0, 0)),
            pl.BlockSpec((NB, D), lambda i: (i, 0)),
            pl.BlockSpec((R, EB), lambda i: (0, i)),
            pl.BlockSpec((EB,), lambda i: (i,)),
        ],
        out_specs=[
            pl.BlockSpec((BS, NB), lambda i: (0, i)),
            pl.BlockSpec((EB,), lambda i: (i,)),
        ],
        out_shape=[
            jax.ShapeDtypeStruct((BS, N), jnp.float32),
            jax.ShapeDtypeStruct((M,), jnp.int32),
        ],
    )(emb, all_emb, r_mask, node_src)

    return _sc_edge_kernel(disT, w, edge_dst, perm, qrel)
